# baseline (speedup 1.0000x reference)
import functools

import jax
import jax.numpy as jnp
from jax import lax
from jax.experimental import pallas as pl
from jax.experimental.pallas import tpu as pltpu
from jax.experimental.pallas import tpu_sc as plsc

_N_KP = 8
_OUT_DIM = 4
_D = _N_KP * _OUT_DIM

_info = plsc.get_sparse_core_info()
_NC, _NS = _info.num_cores, _info.num_subcores
_NW = _NC * _NS

_CHUNK = 128


def _gather_body(idx_hbm, table_hbm, out_hbm, idx_v, rows_v, sem, *, n_chunks):
    wid = lax.axis_index("s") * _NC + lax.axis_index("c")
    pltpu.sync_copy(idx_hbm.at[wid], idx_v)
    copies = [
        pltpu.async_copy(table_hbm.at[idx_v.at[j]], rows_v.at[j], sem)
        for j in range(n_chunks)
    ]
    for c in copies:
        c.wait()
    base = wid * (n_chunks * _CHUNK)
    for j in range(n_chunks):
        pltpu.sync_copy(rows_v.at[j], out_hbm.at[pl.ds(base + j * _CHUNK, _CHUNK)])


def kernel(idx, emb_weight):
    b = idx.shape[0]
    assert b % (_NW * _CHUNK) == 0
    n_chunks = b // (_NW * _CHUNK)
    idx32 = idx.astype(jnp.int32).reshape(_NW, n_chunks, _CHUNK)
    mesh = plsc.VectorSubcoreMesh(core_axis_name="c", subcore_axis_name="s")
    k = pl.kernel(
        functools.partial(_gather_body, n_chunks=n_chunks),
        mesh=mesh,
        out_type=jax.ShapeDtypeStruct((b, _D), jnp.float32),
        scratch_types=[
            pltpu.VMEM((n_chunks, _CHUNK), jnp.int32),
            pltpu.VMEM((n_chunks, _CHUNK, _D), jnp.float32),
            pltpu.SemaphoreType.DMA,
        ],
        compiler_params=pltpu.CompilerParams(use_tc_tiling_on_sc=False),
    )
    out = k(idx32, emb_weight)
    return out.reshape(b, _N_KP, _OUT_DIM)


# speedup vs baseline: 1.1076x; 1.1076x over previous
import jax
import jax.numpy as jnp
from jax import lax
from jax.experimental import pallas as pl
from jax.experimental.pallas import tpu as pltpu
from jax.experimental.pallas import tpu_sc as plsc

_info = plsc.get_sparse_core_info()
_NC, _NS = _info.num_cores, _info.num_subcores
_NW = _NC * _NS

_B = 16384
_V = 1000000
_D = 32
_TAIL0 = 999936
_WIN_C = 512
_NWIN = 62
_BATCH = 512
_DUMP = _B
_OUT_ROWS = _B + 16
_OUT_W = 128


def _body(idx_hbm, tab_hbm, tail_hbm, out_hbm,
          idx_chunk, my_idx, my_k, slab0, slab1, tailbuf,
          wp0, wp1, outbuf, stg0, stg1, kv, sem0, sem1, sem_out):
    i32 = jnp.int32
    wid = lax.axis_index("s") * _NC + lax.axis_index("c")
    lo_t = wid * 244 + jnp.minimum(wid, 4)
    ntiles = jnp.where(wid < 4, 245, 244)
    lo_c = lo_t * 128
    hi_c = jnp.where(wid == _NW - 1, _V, (lo_t + ntiles) * 128)
    lanes = lax.iota(i32, 16)

    @pl.when(wid == _NW - 1)
    def _():
        pltpu.sync_copy(tail_hbm, tailbuf)

    def count_body(ci, cnt):
        pltpu.sync_copy(idx_hbm.at[pl.ds(ci * 2048, 2048)], idx_chunk)

        def g_body(g, cnt):
            iv = idx_chunk[pl.ds(g * 16, 16)]
            m = (iv >= lo_c) & (iv < hi_c)
            return cnt + jnp.sum(m.astype(i32))

        return lax.fori_loop(0, 128, g_body, cnt)

    my_cnt = lax.fori_loop(0, 8, count_body, i32(0))

    zeros16 = jnp.zeros((16,), i32)
    def z_body(g, _):
        wp0[pl.ds(g * 16, 16)] = zeros16
        wp1[pl.ds(g * 16, 16)] = zeros16
        return 0
    lax.fori_loop(0, 32, z_body, 0)

    slabs, sems, wps = (slab0, slab1), (sem0, sem1), (wp0, wp1)

    def win_c0(t):
        return pl.multiple_of((lo_t + jnp.minimum(4 * t, ntiles - 4)) * 128, 128)

    nb = (my_cnt + (_BATCH - 1)) // _BATCH

    def batch_body(b, _):
        base = b * _BATCH
        dump = jnp.full((16,), _DUMP, i32)

        def cap_body(ci, cnt):
            pltpu.sync_copy(idx_hbm.at[pl.ds(ci * 2048, 2048)], idx_chunk)

            def g_body(g, cnt):
                iv = idx_chunk[pl.ds(g * 16, 16)]
                m = (iv >= lo_c) & (iv < hi_c)
                mi = m.astype(i32)
                pos = cnt + plsc.cumsum(mi) - mi
                sel = m & (pos >= base) & (pos < base + _BATCH)
                rel = jnp.clip(pos - base, 0, 2 * _BATCH - 1)
                plsc.store_scatter(my_idx, [rel], iv, mask=sel)
                plsc.store_scatter(my_k, [rel], ci * 2048 + g * 16 + lanes,
                                   mask=sel)
                return cnt + jnp.sum(mi)

            return lax.fori_loop(0, 128, g_body, cnt)

        lax.fori_loop(0, 8, cap_body, i32(0))

        vfull = jnp.full((16,), _V, i32)
        cnt_rel = jnp.clip(my_cnt - base, 0, _BATCH)

        def pad_body(g, _):
            plsc.store_scatter(my_idx, [cnt_rel + g * 16 + lanes], vfull)
            return 0
        lax.fori_loop(0, 32, pad_body, 0)

        def kv_body(g, _):
            q = g * 16 + lanes
            plsc.store_scatter(kv, [q >> 6, q & 63], dump)
            return 0
        lax.fori_loop(0, 32, kv_body, 0)

        def scan_win(c0, c1, wpbuf):
            def s_body(g, w):
                iv = my_idx[pl.ds(g * 16, 16)]
                m = (iv >= c0) & (iv < c1)
                mi = m.astype(i32)
                plsc.store_scatter(wpbuf, [w + plsc.cumsum(mi) - mi],
                                   g * 16 + lanes, mask=m)
                return w + jnp.sum(mi)
            return lax.fori_loop(0, 32, s_body, i32(0))

        def extract(wcnt, wpbuf, src, mkidx):
            ng = (wcnt + 15) // 16

            def e_body(g, _):
                gm = (g * 16 + lanes) < wcnt
                pv = wpbuf[pl.ds(g * 16, 16)]
                iv = plsc.load_gather(my_idx, [pv])
                kk = plsc.load_gather(my_k, [pv])
                plsc.store_scatter(kv, [pv >> 6, pv & 63], kk, mask=gm)
                for j in range(_D):
                    jv = jnp.full((16,), j, i32)
                    vals = plsc.load_gather(src, mkidx(iv, jv))
                    plsc.store_scatter(outbuf, [pv, jv], vals, mask=gm)
                return 0

            lax.fori_loop(0, ng, e_body, 0)

        def slab_dma(t, slab, sem):
            c0 = win_c0(t)
            return pltpu.async_copy(tab_hbm.at[:, pl.ds(c0, _WIN_C)], slab, sem)

        def slab_wait(t, slab, sem):
            c0 = win_c0(t)
            pltpu.make_async_copy(tab_hbm.at[:, pl.ds(c0, _WIN_C)],
                                  slab, sem).wait()

        def do_extract(t, wcnt, wpbuf, slab):
            c0 = win_c0(t)
            extract(wcnt, wpbuf, slab,
                    lambda iv, jv: [jv, jnp.clip(iv - c0, 0, _WIN_C - 1)])

        c0_0 = win_c0(0)
        w0 = scan_win(c0_0, c0_0 + _WIN_C, wp0)

        @pl.when(w0 > 0)
        def _():
            slab_dma(0, slab0, sem0)

        def pair_body(u, w_even):
            t_even = 2 * u
            t_odd = 2 * u + 1
            c0o = win_c0(t_odd)
            w_odd = scan_win(c0o, c0o + _WIN_C, wp1)

            @pl.when(w_odd > 0)
            def _():
                slab_dma(t_odd, slab1, sem1)

            @pl.when(w_even > 0)
            def _():
                slab_wait(t_even, slab0, sem0)
                do_extract(t_even, w_even, wp0, slab0)

            t_next = t_even + 2
            c0n = win_c0(jnp.minimum(t_next, _NWIN - 1))
            w_next = scan_win(c0n, c0n + _WIN_C, wp0)
            w_next = jnp.where(t_next < _NWIN, w_next, i32(0))

            @pl.when(w_next > 0)
            def _():
                slab_dma(t_next, slab0, sem0)

            @pl.when(w_odd > 0)
            def _():
                slab_wait(t_odd, slab1, sem1)
                do_extract(t_odd, w_odd, wp1, slab1)

            return w_next

        lax.fori_loop(0, _NWIN // 2, pair_body, w0)

        wtl = scan_win(i32(_TAIL0), i32(_V), wps[1])

        @pl.when(wtl > 0)
        def _():
            extract(wtl, wps[1], tailbuf,
                    lambda iv, jv: [jnp.clip(iv - _TAIL0, 0, _V - _TAIL0 - 1), jv])

        stgs = (stg0, stg1)
        handles = []
        for grp in range(8):
            if grp >= 2:
                handles[grp - 2].wait()
            stg = stgs[grp % 2]

            def rc_body(q, _, grp=grp, stg=stg):
                rv = (q >> 1) * 16 + lanes
                jj = q & 1

                def c_body(h, _):
                    jv = jj * 16 + h
                    vals = plsc.load_gather(outbuf, [grp * 64 + rv,
                                                     jnp.full((16,), jv, i32)])
                    plsc.store_scatter(stg, [rv, jnp.full((16,), jv, i32)], vals)
                    return 0
                lax.fori_loop(0, 16, c_body, 0)
                return 0
            lax.fori_loop(0, 8, rc_body, 0)
            handles.append(
                pltpu.async_copy(stg, out_hbm.at[kv.at[grp]], sem_out))
        handles[6].wait()
        handles[7].wait()
        return 0

    lax.fori_loop(0, nb, batch_body, 0)


def kernel(idx, emb_weight):
    idx32 = idx.astype(jnp.int32)
    tail = lax.slice(emb_weight, (_TAIL0, 0), (_V, _D))
    mesh = plsc.VectorSubcoreMesh(core_axis_name="c", subcore_axis_name="s")
    k = pl.kernel(
        _body,
        mesh=mesh,
        out_type=jax.ShapeDtypeStruct((_OUT_ROWS, _OUT_W), jnp.float32),
        scratch_types=[
            pltpu.VMEM((2048,), jnp.int32),
            pltpu.VMEM((2 * _BATCH,), jnp.int32),
            pltpu.VMEM((2 * _BATCH,), jnp.int32),
            pltpu.VMEM((_D, _WIN_C), jnp.float32),
            pltpu.VMEM((_D, _WIN_C), jnp.float32),
            pltpu.VMEM((_V - _TAIL0, _D), jnp.float32),
            pltpu.VMEM((_BATCH,), jnp.int32),
            pltpu.VMEM((_BATCH,), jnp.int32),
            pltpu.VMEM((_BATCH, _D), jnp.float32),
            pltpu.VMEM((64, _OUT_W), jnp.float32),
            pltpu.VMEM((64, _OUT_W), jnp.float32),
            pltpu.VMEM((8, 64), jnp.int32),
            pltpu.SemaphoreType.DMA,
            pltpu.SemaphoreType.DMA,
            pltpu.SemaphoreType.DMA,
        ],
        compiler_params=pltpu.CompilerParams(
            use_tc_tiling_on_sc=True, needs_layout_passes=False),
    )
    out = k(idx32, emb_weight.T, tail)
    return out[:_B, :_D].reshape(_B, 8, 4)
